# baseline (speedup 1.0000x reference)
import jax
import jax.numpy as jnp
from jax import lax
from jax.experimental import pallas as pl
from jax.experimental.pallas import tpu as pltpu
from jax.experimental.pallas import tpu_sc as plsc

_ENC_DIM = 32
_ENC_DEPTH = 4
_CB_SIZE = 256
_RES = (16, 32, 64, 128)
_N = 131072
_NC, _NS, _L = 2, 16, 16
_NW = _NC * _NS
_PTS = _N // _NW
_CHUNK = 256
_NCHUNK = _PTS // _CHUNK
_G = _CHUNK // _L
_IDXC = 8 * _CHUNK
_QROWS = 2 * _CHUNK
_ODIM = _ENC_DEPTH * _ENC_DIM
_OPAD = _ODIM + 1

_OFFS = tuple((i, j, k) for i in (0, 1) for j in (0, 1) for k in (0, 1))


def _corner_setup(x_v, y_v, z_v, off, res):
    res_f = float(res)
    bases = []
    fracs = []
    for ref in (x_v, y_v, z_v):
        p = ref[pl.ds(off, _L)]
        p = jnp.minimum(jnp.maximum(p, 0.0), 1.0) * res_f
        b = jnp.minimum(p.astype(jnp.int32), res - 1)
        bases.append(b)
        fracs.append(p - b.astype(jnp.float32))
    return bases, fracs


def _sc_body(x_hbm, y_hbm, z_hbm, cbt_hbm, t0, t1, t2q, t3q, out_hbm,
             cb_v, x_v, y_v, z_v, tb0_v, tb1_v, idx_v, ent_q, obuf,
             sem_a, sem_b):
    stbls = (t2q, t3q)
    wid = lax.axis_index("s") * _NC + lax.axis_index("c")
    base = wid * _PTS

    pltpu.sync_copy(cbt_hbm, cb_v)
    pltpu.sync_copy(t0.at[pl.ds(0, (_RES[0] + 1) ** 3)], tb0_v)
    pltpu.sync_copy(t1.at[pl.ds(0, (_RES[1] + 1) ** 3)], tb1_v)
    pltpu.sync_copy(x_hbm.at[pl.ds(base, _PTS)], x_v)
    pltpu.sync_copy(y_hbm.at[pl.ds(base, _PTS)], y_v)
    pltpu.sync_copy(z_hbm.at[pl.ds(base, _PTS)], z_v)

    lane = lax.iota(jnp.int32, _L)
    half = 2 * _QROWS

    def phase1(pbase, boff):
        for sl, res in enumerate(_RES[2:]):
            s = res + 1

            @plsc.parallel_loop(0, _G)
            def p1(g, _sl=sl, _s=s, _res=res):
                off = pbase + g * _L
                (bx, by, bz), _ = _corner_setup(x_v, y_v, z_v, off, _res)
                lin0 = bx * (_s * _s) + by * _s + bz
                gb = boff + _sl * _QROWS + g * _L
                idx_v[pl.ds(gb, _L)] = lin0
                idx_v[pl.ds(gb + _CHUNK, _L)] = lin0 + _s * _s

    def fire(boff, sem):
        for sl in range(2):
            pltpu.async_copy(
                stbls[sl].at[idx_v.at[pl.ds(boff + sl * _QROWS, _QROWS)]],
                ent_q.at[pl.ds(boff + sl * _QROWS, _QROWS)], sem)

    def drain(boff, sem):
        for sl in range(2):
            pltpu.make_async_copy(
                stbls[sl].at[idx_v.at[pl.ds(boff + sl * _QROWS, _QROWS)]],
                ent_q.at[pl.ds(boff + sl * _QROWS, _QROWS)], sem).wait()

    def phase2(pbase, boff):
        for l, res in enumerate(_RES):
            s = res + 1

            @plsc.parallel_loop(0, _G)
            def p2(g, _l=l, _s=s, _res=res):
                off = pbase + g * _L
                (bx, by, bz), (fx, fy, fz) = _corner_setup(
                    x_v, y_v, z_v, off, _res)
                wx = (1.0 - fx, fx)
                wy = (1.0 - fy, fy)
                wz = (1.0 - fz, fz)
                es = []
                wp = []
                if _l < 2:
                    tbl_v = tb0_v if _l == 0 else tb1_v
                    lin0 = bx * (_s * _s) + by * _s + bz
                else:
                    gb = boff + (_l - 2) * _QROWS + g * _L
                    qw = (ent_q[pl.ds(gb, _L)],
                          ent_q[pl.ds(gb + _CHUNK, _L)])
                for ci, (i, j, k) in enumerate(_OFFS):
                    w = (wx[i] * wy[j]) * wz[k]
                    wp.append(plsc.pack(w, w,
                                        format=plsc.PackFormat.INTERLEAVED))
                    if _l < 2:
                        linc = lin0 + (i * _s * _s + j * _s + k)
                        es.append(plsc.load_gather(tbl_v, [linc]))
                    else:
                        sh = 8 * (2 * j + k)
                        e = qw[i]
                        if sh:
                            e = lax.shift_right_logical(
                                e, jnp.full((_L,), sh, jnp.int32))
                        if sh < 24:
                            e = e & jnp.int32(0xFF)
                        es.append(e)
                pv = (lane + g * _L) * _OPAD + _l * _ENC_DIM
                for t in range(_ENC_DIM // 2):
                    cbref = cb_v.at[pl.ds(
                        (_l * (_ENC_DIM // 2) + t) * _CB_SIZE, _CB_SIZE)]
                    p = [
                        wp[ci] * plsc.bitcast(
                            plsc.load_gather(cbref, [es[ci]]), jnp.bfloat16)
                        for ci in range(8)
                    ]
                    q = [p[0] + p[1], p[2] + p[3], p[4] + p[5], p[6] + p[7]]
                    acc = (q[0] + q[1]) + (q[2] + q[3])
                    flo, fhi = plsc.unpack(
                        acc, format=plsc.PackFormat.INTERLEAVED)
                    plsc.store_scatter(obuf, [pv + 2 * t], flo)
                    plsc.store_scatter(obuf, [pv + 2 * t + 1], fhi)

        pltpu.sync_copy(
            obuf, out_hbm.at[pl.ds((base + pbase) * _OPAD, _CHUNK * _OPAD)])

    phase1(0, 0)
    fire(0, sem_a)

    def pair_body(j, carry):
        ch0 = 2 * j * _CHUNK
        ch1 = ch0 + _CHUNK
        phase1(ch1, half)
        fire(half, sem_b)
        drain(0, sem_a)
        phase2(ch0, 0)
        nxt = (ch1 + _CHUNK) & (_PTS - 1)
        phase1(nxt, 0)
        fire(0, sem_a)
        drain(half, sem_b)
        phase2(ch1, half)
        return carry

    lax.fori_loop(0, _NCHUNK // 2, pair_body, 0)
    drain(0, sem_a)


def kernel(inp, codebook, indices):
    xs = inp[:, 0]
    ys = inp[:, 1]
    zs = inp[:, 2]
    cbb = codebook.astype(jnp.bfloat16)
    cbb = jnp.transpose(cbb, (0, 2, 1))
    cbb = cbb.reshape(_ENC_DEPTH, _ENC_DIM // 2, 2, _CB_SIZE)
    cbb = jnp.transpose(cbb, (0, 1, 3, 2))
    cbt = jax.lax.bitcast_convert_type(cbb, jnp.int32).reshape(-1)
    tbls = [indices[0], indices[1]]
    for l in (2, 3):
        t = indices[l]
        s = _RES[l] + 1
        tbls.append(t
                    | (jnp.roll(t, -1) << 8)
                    | (jnp.roll(t, -s) << 16)
                    | (jnp.roll(t, -(s + 1)) << 24))

    mesh = plsc.VectorSubcoreMesh(core_axis_name="c", subcore_axis_name="s")
    run = pl.kernel(
        _sc_body,
        out_type=jax.ShapeDtypeStruct((_N * _OPAD,), jnp.float32),
        mesh=mesh,
        compiler_params=pltpu.CompilerParams(needs_layout_passes=False),
        scratch_types=[
            pltpu.VMEM((_ENC_DEPTH * (_ENC_DIM // 2) * _CB_SIZE,),
                       jnp.int32),
            pltpu.VMEM((_PTS,), jnp.float32),
            pltpu.VMEM((_PTS,), jnp.float32),
            pltpu.VMEM((_PTS,), jnp.float32),
            pltpu.VMEM(((_RES[0] + 1) ** 3,), jnp.int32),
            pltpu.VMEM(((_RES[1] + 1) ** 3,), jnp.int32),
            pltpu.VMEM((2 * 2 * _QROWS,), jnp.int32),
            pltpu.VMEM((2 * 2 * _QROWS,), jnp.int32),
            pltpu.VMEM((_CHUNK * _OPAD,), jnp.float32),
            pltpu.SemaphoreType.DMA,
            pltpu.SemaphoreType.DMA,
        ],
    )
    out = run(xs, ys, zs, cbt, *tbls)
    return out.reshape(_N, _OPAD)[:, :_ODIM]


# speedup vs baseline: 1.0581x; 1.0581x over previous
import jax
import jax.numpy as jnp
from jax import lax
from jax.experimental import pallas as pl
from jax.experimental.pallas import tpu as pltpu
from jax.experimental.pallas import tpu_sc as plsc

_ENC_DIM = 32
_ENC_DEPTH = 4
_CB_SIZE = 256
_RES = (16, 32, 64, 128)
_N = 131072
_NC, _NS, _L = 2, 16, 16
_NW = _NC * _NS
_PTS = _N // _NW
_CHUNK = 256
_NCHUNK = _PTS // _CHUNK
_G = _CHUNK // _L
_IDXC = 8 * _CHUNK
_QROWS = 2 * _CHUNK
_ODIM = _ENC_DEPTH * _ENC_DIM
_OPAD = _ODIM

_OFFS = tuple((i, j, k) for i in (0, 1) for j in (0, 1) for k in (0, 1))


def _corner_setup(x_v, y_v, z_v, off, res):
    res_f = float(res)
    bases = []
    fracs = []
    for ref in (x_v, y_v, z_v):
        p = ref[pl.ds(off, _L)]
        p = jnp.minimum(jnp.maximum(p, 0.0), 1.0) * res_f
        b = jnp.minimum(p.astype(jnp.int32), res - 1)
        bases.append(b)
        fracs.append(p - b.astype(jnp.float32))
    return bases, fracs


def _sc_body(x_hbm, y_hbm, z_hbm, cbt_hbm, t0, t1, t2q, t3q, out_hbm,
             cb_v, x_v, y_v, z_v, tb0_v, tb1_v, idx_v, ent_q, obuf,
             sem_a, sem_b):
    stbls = (t2q, t3q)
    wid = lax.axis_index("s") * _NC + lax.axis_index("c")
    base = wid * _PTS

    pltpu.sync_copy(cbt_hbm, cb_v)
    pltpu.sync_copy(t0.at[pl.ds(0, (_RES[0] + 1) ** 3)], tb0_v)
    pltpu.sync_copy(t1.at[pl.ds(0, (_RES[1] + 1) ** 3)], tb1_v)
    pltpu.sync_copy(x_hbm.at[pl.ds(base, _PTS)], x_v)
    pltpu.sync_copy(y_hbm.at[pl.ds(base, _PTS)], y_v)
    pltpu.sync_copy(z_hbm.at[pl.ds(base, _PTS)], z_v)

    lane = lax.iota(jnp.int32, _L)
    half = 2 * _QROWS

    def phase1(pbase, boff):
        for sl, res in enumerate(_RES[2:]):
            s = res + 1

            @plsc.parallel_loop(0, _G)
            def p1(g, _sl=sl, _s=s, _res=res):
                off = pbase + g * _L
                (bx, by, bz), _ = _corner_setup(x_v, y_v, z_v, off, _res)
                lin0 = bx * (_s * _s) + by * _s + bz
                gb = boff + _sl * _QROWS + g * _L
                idx_v[pl.ds(gb, _L)] = lin0
                idx_v[pl.ds(gb + _CHUNK, _L)] = lin0 + _s * _s

    def fire(boff, sem):
        for sl in range(2):
            pltpu.async_copy(
                stbls[sl].at[idx_v.at[pl.ds(boff + sl * _QROWS, _QROWS)]],
                ent_q.at[pl.ds(boff + sl * _QROWS, _QROWS)], sem)

    def drain(boff, sem):
        for sl in range(2):
            pltpu.make_async_copy(
                stbls[sl].at[idx_v.at[pl.ds(boff + sl * _QROWS, _QROWS)]],
                ent_q.at[pl.ds(boff + sl * _QROWS, _QROWS)], sem).wait()

    def phase2(pbase, boff):
        for l, res in enumerate(_RES):
            s = res + 1

            @plsc.parallel_loop(0, _G)
            def p2(g, _l=l, _s=s, _res=res):
                off = pbase + g * _L
                (bx, by, bz), (fx, fy, fz) = _corner_setup(
                    x_v, y_v, z_v, off, _res)
                wx = (1.0 - fx, fx)
                wy = (1.0 - fy, fy)
                wz = (1.0 - fz, fz)
                es = []
                wp = []
                if _l < 2:
                    tbl_v = tb0_v if _l == 0 else tb1_v
                    lin0 = bx * (_s * _s) + by * _s + bz
                else:
                    gb = boff + (_l - 2) * _QROWS + g * _L
                    qw = (ent_q[pl.ds(gb, _L)],
                          ent_q[pl.ds(gb + _CHUNK, _L)])
                for ci, (i, j, k) in enumerate(_OFFS):
                    w = (wx[i] * wy[j]) * wz[k]
                    wp.append(plsc.pack(w, w,
                                        format=plsc.PackFormat.INTERLEAVED))
                    if _l < 2:
                        linc = lin0 + (i * _s * _s + j * _s + k)
                        es.append(plsc.load_gather(tbl_v, [linc]))
                    else:
                        sh = 8 * (2 * j + k)
                        e = qw[i]
                        if sh:
                            e = lax.shift_right_logical(
                                e, jnp.full((_L,), sh, jnp.int32))
                        if sh < 24:
                            e = e & jnp.int32(0xFF)
                        es.append(e)
                pv = (lane + g * _L) * _OPAD + _l * _ENC_DIM
                for t in range(_ENC_DIM // 2):
                    cbref = cb_v.at[pl.ds(
                        (_l * (_ENC_DIM // 2) + t) * _CB_SIZE, _CB_SIZE)]
                    p = [
                        wp[ci] * plsc.bitcast(
                            plsc.load_gather(cbref, [es[ci]]), jnp.bfloat16)
                        for ci in range(8)
                    ]
                    q = [p[0] + p[1], p[2] + p[3], p[4] + p[5], p[6] + p[7]]
                    acc = (q[0] + q[1]) + (q[2] + q[3])
                    flo, fhi = plsc.unpack(
                        acc, format=plsc.PackFormat.INTERLEAVED)
                    plsc.store_scatter(obuf, [pv + 2 * t], flo)
                    plsc.store_scatter(obuf, [pv + 2 * t + 1], fhi)

        pltpu.sync_copy(
            obuf, out_hbm.at[pl.ds((base + pbase) * _OPAD, _CHUNK * _OPAD)])

    phase1(0, 0)
    fire(0, sem_a)

    def pair_body(j, carry):
        ch0 = 2 * j * _CHUNK
        ch1 = ch0 + _CHUNK
        phase1(ch1, half)
        fire(half, sem_b)
        drain(0, sem_a)
        phase2(ch0, 0)
        nxt = (ch1 + _CHUNK) & (_PTS - 1)
        phase1(nxt, 0)
        fire(0, sem_a)
        drain(half, sem_b)
        phase2(ch1, half)
        return carry

    lax.fori_loop(0, _NCHUNK // 2, pair_body, 0)
    drain(0, sem_a)


def kernel(inp, codebook, indices):
    xs = inp[:, 0]
    ys = inp[:, 1]
    zs = inp[:, 2]
    cbb = codebook.astype(jnp.bfloat16)
    cbb = jnp.transpose(cbb, (0, 2, 1))
    cbb = cbb.reshape(_ENC_DEPTH, _ENC_DIM // 2, 2, _CB_SIZE)
    cbb = jnp.transpose(cbb, (0, 1, 3, 2))
    cbt = jax.lax.bitcast_convert_type(cbb, jnp.int32).reshape(-1)
    tbls = [indices[0], indices[1]]
    for l in (2, 3):
        t = indices[l]
        s = _RES[l] + 1
        tbls.append(t
                    | (jnp.roll(t, -1) << 8)
                    | (jnp.roll(t, -s) << 16)
                    | (jnp.roll(t, -(s + 1)) << 24))

    mesh = plsc.VectorSubcoreMesh(core_axis_name="c", subcore_axis_name="s")
    run = pl.kernel(
        _sc_body,
        out_type=jax.ShapeDtypeStruct((_N * _OPAD,), jnp.float32),
        mesh=mesh,
        compiler_params=pltpu.CompilerParams(needs_layout_passes=False),
        scratch_types=[
            pltpu.VMEM((_ENC_DEPTH * (_ENC_DIM // 2) * _CB_SIZE,),
                       jnp.int32),
            pltpu.VMEM((_PTS,), jnp.float32),
            pltpu.VMEM((_PTS,), jnp.float32),
            pltpu.VMEM((_PTS,), jnp.float32),
            pltpu.VMEM(((_RES[0] + 1) ** 3,), jnp.int32),
            pltpu.VMEM(((_RES[1] + 1) ** 3,), jnp.int32),
            pltpu.VMEM((2 * 2 * _QROWS,), jnp.int32),
            pltpu.VMEM((2 * 2 * _QROWS,), jnp.int32),
            pltpu.VMEM((_CHUNK * _OPAD,), jnp.float32),
            pltpu.SemaphoreType.DMA,
            pltpu.SemaphoreType.DMA,
        ],
    )
    out = run(xs, ys, zs, cbt, *tbls)
    return out.reshape(_N, _OPAD)[:, :_ODIM]


# speedup vs baseline: 1.1743x; 1.1098x over previous
import jax
import jax.numpy as jnp
from jax import lax
from jax.experimental import pallas as pl
from jax.experimental.pallas import tpu as pltpu
from jax.experimental.pallas import tpu_sc as plsc

_ENC_DIM = 32
_ENC_DEPTH = 4
_CB_SIZE = 256
_RES = (16, 32, 64, 128)
_N = 131072
_NC, _NS, _L = 2, 16, 16
_NW = _NC * _NS
_PTS = _N // _NW
_CHUNK = 256
_NCHUNK = _PTS // _CHUNK
_G = _CHUNK // _L
_IDXC = 8 * _CHUNK
_QROWS = 2 * _CHUNK
_ODIM = _ENC_DEPTH * _ENC_DIM
_OPAD = _ODIM

_OFFS = tuple((i, j, k) for i in (0, 1) for j in (0, 1) for k in (0, 1))


def _corner_setup(x_v, y_v, z_v, off, res):
    res_f = float(res)
    bases = []
    fracs = []
    for ref in (x_v, y_v, z_v):
        p = ref[pl.ds(off, _L)]
        p = jnp.minimum(jnp.maximum(p, 0.0), 1.0) * res_f
        b = jnp.minimum(p.astype(jnp.int32), res - 1)
        bases.append(b)
        fracs.append(p - b.astype(jnp.float32))
    return bases, fracs


def _sc_body(x_hbm, y_hbm, z_hbm, cbt_hbm, t0, t1, t2q, t3q, out_hbm,
             cb_v, x_v, y_v, z_v, tb0_v, tb1_v, idx_v, ent_q, obuf,
             sem_a, sem_b):
    stbls = (t2q, t3q)
    wid = lax.axis_index("s") * _NC + lax.axis_index("c")
    base = wid * _PTS

    pltpu.sync_copy(cbt_hbm, cb_v)
    pltpu.sync_copy(t0.at[pl.ds(0, (_RES[0] + 1) ** 3)], tb0_v)
    pltpu.sync_copy(t1.at[pl.ds(0, (_RES[1] + 1) ** 3)], tb1_v)
    pltpu.sync_copy(x_hbm.at[pl.ds(base, _PTS)], x_v)
    pltpu.sync_copy(y_hbm.at[pl.ds(base, _PTS)], y_v)
    pltpu.sync_copy(z_hbm.at[pl.ds(base, _PTS)], z_v)

    lane = lax.iota(jnp.int32, _L)
    half = 2 * _IDXC

    def phase1(pbase, boff):
        for sl, res in enumerate(_RES[2:]):
            s = res + 1

            @plsc.parallel_loop(0, _G)
            def p1(g, _sl=sl, _s=s, _res=res):
                off = pbase + g * _L
                (bx, by, bz), _ = _corner_setup(x_v, y_v, z_v, off, _res)
                lin0 = bx * (_s * _s) + by * _s + bz
                gb = boff + _sl * _IDXC + g * (8 * _L)
                for ci, (i, j, k) in enumerate(_OFFS):
                    linc = lin0 + (i * _s * _s + j * _s + k)
                    idx_v[pl.ds(gb + ci * _L, _L)] = linc

    def fire(boff, sem):
        for sl in range(2):
            pltpu.async_copy(
                stbls[sl].at[idx_v.at[pl.ds(boff + sl * _IDXC, _IDXC)]],
                ent_q.at[pl.ds(boff + sl * _IDXC, _IDXC)], sem)

    def drain(boff, sem):
        for sl in range(2):
            pltpu.make_async_copy(
                stbls[sl].at[idx_v.at[pl.ds(boff + sl * _IDXC, _IDXC)]],
                ent_q.at[pl.ds(boff + sl * _IDXC, _IDXC)], sem).wait()

    def phase2(pbase, boff):
        for l, res in enumerate(_RES):
            s = res + 1

            @plsc.parallel_loop(0, _G)
            def p2(g, _l=l, _s=s, _res=res):
                off = pbase + g * _L
                (bx, by, bz), (fx, fy, fz) = _corner_setup(
                    x_v, y_v, z_v, off, _res)
                wx = (1.0 - fx, fx)
                wy = (1.0 - fy, fy)
                wz = (1.0 - fz, fz)
                es = []
                wp = []
                if _l < 2:
                    tbl_v = tb0_v if _l == 0 else tb1_v
                    lin0 = bx * (_s * _s) + by * _s + bz
                else:
                    gb = boff + (_l - 2) * _IDXC + g * (8 * _L)
                for ci, (i, j, k) in enumerate(_OFFS):
                    w = (wx[i] * wy[j]) * wz[k]
                    wp.append(plsc.pack(w, w,
                                        format=plsc.PackFormat.INTERLEAVED))
                    if _l < 2:
                        linc = lin0 + (i * _s * _s + j * _s + k)
                        es.append(plsc.load_gather(tbl_v, [linc]))
                    else:
                        es.append(ent_q[pl.ds(gb + ci * _L, _L)])
                pv = (lane + g * _L) * _OPAD + _l * _ENC_DIM
                for t in range(_ENC_DIM // 2):
                    cbref = cb_v.at[pl.ds(
                        (_l * (_ENC_DIM // 2) + t) * _CB_SIZE, _CB_SIZE)]
                    p = [
                        wp[ci] * plsc.bitcast(
                            plsc.load_gather(cbref, [es[ci]]), jnp.bfloat16)
                        for ci in range(8)
                    ]
                    q = [p[0] + p[1], p[2] + p[3], p[4] + p[5], p[6] + p[7]]
                    acc = (q[0] + q[1]) + (q[2] + q[3])
                    flo, fhi = plsc.unpack(
                        acc, format=plsc.PackFormat.INTERLEAVED)
                    plsc.store_scatter(obuf, [pv + 2 * t], flo)
                    plsc.store_scatter(obuf, [pv + 2 * t + 1], fhi)

        pltpu.sync_copy(
            obuf, out_hbm.at[pl.ds((base + pbase) * _OPAD, _CHUNK * _OPAD)])

    phase1(0, 0)
    fire(0, sem_a)

    def pair_body(j, carry):
        ch0 = 2 * j * _CHUNK
        ch1 = ch0 + _CHUNK
        phase1(ch1, half)
        fire(half, sem_b)
        drain(0, sem_a)
        phase2(ch0, 0)
        nxt = (ch1 + _CHUNK) & (_PTS - 1)
        phase1(nxt, 0)
        fire(0, sem_a)
        drain(half, sem_b)
        phase2(ch1, half)
        return carry

    lax.fori_loop(0, _NCHUNK // 2, pair_body, 0)
    drain(0, sem_a)


def kernel(inp, codebook, indices):
    xs = inp[:, 0]
    ys = inp[:, 1]
    zs = inp[:, 2]
    cbb = codebook.astype(jnp.bfloat16)
    cbb = jnp.transpose(cbb, (0, 2, 1))
    cbb = cbb.reshape(_ENC_DEPTH, _ENC_DIM // 2, 2, _CB_SIZE)
    cbb = jnp.transpose(cbb, (0, 1, 3, 2))
    cbt = jax.lax.bitcast_convert_type(cbb, jnp.int32).reshape(-1)
    tbls = [indices[l] for l in range(_ENC_DEPTH)]

    mesh = plsc.VectorSubcoreMesh(core_axis_name="c", subcore_axis_name="s")
    run = pl.kernel(
        _sc_body,
        out_type=jax.ShapeDtypeStruct((_N * _OPAD,), jnp.float32),
        mesh=mesh,
        compiler_params=pltpu.CompilerParams(needs_layout_passes=False),
        scratch_types=[
            pltpu.VMEM((_ENC_DEPTH * (_ENC_DIM // 2) * _CB_SIZE,),
                       jnp.int32),
            pltpu.VMEM((_PTS,), jnp.float32),
            pltpu.VMEM((_PTS,), jnp.float32),
            pltpu.VMEM((_PTS,), jnp.float32),
            pltpu.VMEM(((_RES[0] + 1) ** 3,), jnp.int32),
            pltpu.VMEM(((_RES[1] + 1) ** 3,), jnp.int32),
            pltpu.VMEM((2 * 2 * _IDXC,), jnp.int32),
            pltpu.VMEM((2 * 2 * _IDXC,), jnp.int32),
            pltpu.VMEM((_CHUNK * _OPAD,), jnp.float32),
            pltpu.SemaphoreType.DMA,
            pltpu.SemaphoreType.DMA,
        ],
    )
    out = run(xs, ys, zs, cbt, *tbls)
    return out.reshape(_N, _OPAD)[:, :_ODIM]
